# Initial kernel scaffold; baseline (speedup 1.0000x reference)
import jax
import jax.numpy as jnp
from jax.experimental import pallas as pl


def kernel(x):
    raise NotImplementedError("write your pallas kernel here")


# speedup vs baseline: 17.6694x; 17.6694x over previous
import functools

import jax
import jax.numpy as jnp
from jax import lax
from jax.experimental import pallas as pl

_N = 32768
_ROWS_PER_BLOCK = 16
_INT_MIN = -(2**31)


def _kwta_block(m, x_ref, o_ref):
    xb = x_ref[...]
    b = lax.bitcast_convert_type(xb, jnp.int32)
    skey = b ^ ((b >> 31) & jnp.int32(0x7FFFFFFF))

    rows = xb.shape[0]
    prefix0 = jnp.full((rows, 1), _INT_MIN, dtype=jnp.int32)

    def step(t, prefix):
        bit = lax.shift_left(jnp.int32(1), jnp.int32(31) - t)
        cand = prefix + bit
        cnt = jnp.sum((skey >= cand).astype(jnp.int32), axis=1, keepdims=True)
        return jnp.where(cnt >= m, cand, prefix)

    prefix = lax.fori_loop(0, 32, step, prefix0, unroll=True)

    thr_bits = jnp.where(prefix >= 0, prefix, prefix ^ jnp.int32(0x7FFFFFFF))
    thresh = lax.bitcast_convert_type(thr_bits, jnp.float32) + jnp.float32(1e-8)
    y = jnp.maximum(xb - thresh, 0.0)
    s = jnp.sum(y, axis=1, keepdims=True)
    o_ref[...] = y / (s + jnp.float32(1e-8))


def kernel(x):
    n_rows, n = x.shape
    k = int(round(n * 0.2))
    m = k - 1
    grid = n_rows // _ROWS_PER_BLOCK
    return pl.pallas_call(
        functools.partial(_kwta_block, m),
        grid=(grid,),
        in_specs=[pl.BlockSpec((_ROWS_PER_BLOCK, n), lambda i: (i, 0))],
        out_specs=pl.BlockSpec((_ROWS_PER_BLOCK, n), lambda i: (i, 0)),
        out_shape=jax.ShapeDtypeStruct((n_rows, n), jnp.float32),
    )(x)
